# Initial kernel scaffold; baseline (speedup 1.0000x reference)
import jax
import jax.numpy as jnp
from jax.experimental import pallas as pl


def kernel(similarities):
    raise NotImplementedError("write your pallas kernel here")


# speedup vs baseline: 9.5826x; 9.5826x over previous
import jax
import jax.numpy as jnp
from jax.experimental import pallas as pl
from jax.experimental.pallas import tpu as pltpu

_K = 5
_CHUNK = 2048
_DBLK = 2048


def _topk_body(x_ref, idx_ref):
    B, D = x_ref.shape
    nch = D // _CHUNK
    sels = []
    for k in range(_K):
        best_v = jnp.full((B, 1), -jnp.inf, dtype=jnp.float32)
        best_i = jnp.zeros((B, 1), dtype=jnp.int32)
        for c in range(nch):
            v = x_ref[:, c * _CHUNK:(c + 1) * _CHUNK]
            col = jax.lax.broadcasted_iota(jnp.int32, (B, _CHUNK), 1) + c * _CHUNK
            for j in range(k):
                v = jnp.where(col == sels[j], -jnp.inf, v)
            cm = jnp.max(v, axis=1, keepdims=True)
            ci = jnp.min(jnp.where(v == cm, col, D), axis=1, keepdims=True)
            upd = cm > best_v
            best_v = jnp.where(upd, cm, best_v)
            best_i = jnp.where(upd, ci, best_i)
        sels.append(best_i)
    idx_ref[...] = jnp.concatenate(sels, axis=1)


def _onehot_body(idx_ref, out_ref):
    i = pl.program_id(0)
    BK, dblk = out_ref.shape
    idxv = idx_ref[...]
    col = jax.lax.broadcasted_iota(jnp.int32, (BK, dblk), 1) + i * dblk
    out_ref[...] = jnp.where(col == idxv, 1.0, 0.0).astype(jnp.float32)


def kernel(similarities):
    B, D = similarities.shape
    idx = pl.pallas_call(
        _topk_body,
        out_shape=jax.ShapeDtypeStruct((B, _K), jnp.int32),
    )(similarities)

    flat = idx.reshape(B * _K, 1)
    oh = pl.pallas_call(
        _onehot_body,
        grid=(D // _DBLK,),
        in_specs=[pl.BlockSpec((B * _K, 1), lambda i: (0, 0))],
        out_specs=pl.BlockSpec((B * _K, _DBLK), lambda i: (0, i)),
        out_shape=jax.ShapeDtypeStruct((B * _K, D), jnp.float32),
        compiler_params=pltpu.CompilerParams(
            dimension_semantics=("arbitrary",),
        ),
    )(flat)
    return idx, oh.reshape(B, _K, D)


# speedup vs baseline: 16.8177x; 1.7550x over previous
import jax
import jax.numpy as jnp
from jax.experimental import pallas as pl
from jax.experimental.pallas import tpu as pltpu

_K = 5
_CHUNK = 2048
_DBLK = 2048


def _topk_body(x_ref, idx_ref):
    B, D = x_ref.shape
    nch = D // _CHUNK
    sels = []
    for k in range(_K):
        best_v = jnp.full((B, 1), -jnp.inf, dtype=jnp.float32)
        best_i = jnp.zeros((B, 1), dtype=jnp.int32)
        for c in range(nch):
            v = x_ref[:, c * _CHUNK:(c + 1) * _CHUNK]
            col = jax.lax.broadcasted_iota(jnp.int32, (B, _CHUNK), 1) + c * _CHUNK
            for j in range(k):
                v = jnp.where(col == sels[j], -jnp.inf, v)
            cm = jnp.max(v, axis=1, keepdims=True)
            ci = jnp.min(jnp.where(v == cm, col, D), axis=1, keepdims=True)
            upd = cm > best_v
            best_v = jnp.where(upd, cm, best_v)
            best_i = jnp.where(upd, ci, best_i)
        sels.append(best_i)
    idx_ref[...] = jnp.concatenate(sels, axis=1)


def _onehot_body(idx_ref, out_ref):
    i = pl.program_id(0)
    B, K, dblk = out_ref.shape
    idxv = idx_ref[...][:, :, None]
    col = jax.lax.broadcasted_iota(jnp.int32, (B, K, dblk), 2) + i * dblk
    out_ref[...] = jnp.where(col == idxv, 1.0, 0.0).astype(jnp.float32)


def kernel(similarities):
    B, D = similarities.shape
    idx = pl.pallas_call(
        _topk_body,
        out_shape=jax.ShapeDtypeStruct((B, _K), jnp.int32),
    )(similarities)

    oh = pl.pallas_call(
        _onehot_body,
        grid=(D // _DBLK,),
        in_specs=[pl.BlockSpec((B, _K), lambda i: (0, 0))],
        out_specs=pl.BlockSpec((B, _K, _DBLK), lambda i: (0, 0, i)),
        out_shape=jax.ShapeDtypeStruct((B, _K, D), jnp.float32),
        compiler_params=pltpu.CompilerParams(
            dimension_semantics=("arbitrary",),
        ),
    )(idx)
    return idx, oh
